# Initial kernel scaffold; baseline (speedup 1.0000x reference)
import functools

import jax
import jax.numpy as jnp
from jax import lax
from jax.experimental import pallas as pl
from jax.experimental.pallas import tpu as pltpu
from jax.experimental.pallas import tpu_sc as plsc

_LANES = 16
_NUM_CORES = 2
_NUM_SUBCORES = 16
_NW = _NUM_CORES * _NUM_SUBCORES
_CHUNK = 128
_PAD_ID = 0


def _make_gather(total_rows: int, d: int):
  assert total_rows % (_NW * _CHUNK) == 0
  rows_per_w = total_rows // _NW
  steps = rows_per_w // _CHUNK
  mesh = plsc.VectorSubcoreMesh(core_axis_name="c", subcore_axis_name="s")

  @functools.partial(
      pl.kernel,
      mesh=mesh,
      out_type=jax.ShapeDtypeStruct((total_rows, d), jnp.float32),
      scratch_types=[
          pltpu.VMEM((_CHUNK,), jnp.int32),
          pltpu.VMEM((_CHUNK, d), jnp.float32),
          pltpu.SemaphoreType.DMA,
      ],
  )
  def gather_kernel(table_hbm, idx_hbm, out_hbm, idx_v, rows_v, sem):
    wid = lax.axis_index("s") * _NUM_CORES + lax.axis_index("c")
    base = wid * rows_per_w

    def step(s, carry):
      off = base + s * _CHUNK
      pltpu.sync_copy(idx_hbm.at[pl.ds(off, _CHUNK)], idx_v)
      pltpu.async_copy(table_hbm.at[idx_v], rows_v, sem).wait()
      zeros = jnp.zeros((_LANES,), jnp.float32)
      for g in range(_CHUNK // _LANES):
        idx16 = idx_v[pl.ds(g * _LANES, _LANES)]
        m = idx16 == _PAD_ID

        @pl.when(jnp.any(m))
        def _zero_rows():
          rid = lax.iota(jnp.int32, _LANES) + g * _LANES
          for j in range(d):
            cid = jnp.full((_LANES,), j, jnp.int32)
            plsc.store_scatter(rows_v, [rid, cid], zeros, mask=m)

      pltpu.sync_copy(rows_v, out_hbm.at[pl.ds(off, _CHUNK)])
      return carry

    lax.fori_loop(0, steps, step, 0)

  return gather_kernel


def kernel(indices, table):
  b, l = indices.shape
  v, d = table.shape
  idx = indices.reshape(-1).astype(jnp.int32)
  out = _make_gather(b * l, d)(table, idx)
  return out.reshape(b, l, d)


# speedup vs baseline: 1.5734x; 1.5734x over previous
import functools

import jax
import jax.numpy as jnp
from jax import lax
from jax.experimental import pallas as pl
from jax.experimental.pallas import tpu as pltpu
from jax.experimental.pallas import tpu_sc as plsc

_LANES = 16
_NUM_CORES = 2
_NUM_SUBCORES = 16
_NW = _NUM_CORES * _NUM_SUBCORES
_CHUNK = 128
_PAD_ID = 0


def _make_gather(total_rows: int, d: int):
  assert total_rows % (_NW * _CHUNK) == 0
  rows_per_w = total_rows // _NW
  steps = rows_per_w // _CHUNK
  mesh = plsc.VectorSubcoreMesh(core_axis_name="c", subcore_axis_name="s")

  @functools.partial(
      pl.kernel,
      mesh=mesh,
      out_type=jax.ShapeDtypeStruct((total_rows, d), jnp.float32),
      scratch_types=[
          pltpu.VMEM((_CHUNK,), jnp.int32),
          pltpu.VMEM((_CHUNK, d), jnp.float32),
          pltpu.SemaphoreType.DMA,
      ],
      compiler_params=pltpu.CompilerParams(use_tc_tiling_on_sc=False),
  )
  def gather_kernel(table_hbm, idx_hbm, out_hbm, idx_v, rows_v, sem):
    wid = lax.axis_index("s") * _NUM_CORES + lax.axis_index("c")
    base = wid * rows_per_w

    def step(s, carry):
      off = base + s * _CHUNK
      pltpu.sync_copy(idx_hbm.at[pl.ds(off, _CHUNK)], idx_v)
      pltpu.async_copy(table_hbm.at[idx_v], rows_v, sem).wait()
      izeros = jnp.zeros((_LANES,), jnp.int32)
      ones = jnp.ones((_LANES,), jnp.int32)
      acc = izeros
      for g in range(_CHUNK // _LANES):
        idx16 = idx_v[pl.ds(g * _LANES, _LANES)]
        acc = acc | jnp.where(idx16 == _PAD_ID, ones, izeros)
      lane = lax.iota(jnp.int32, _LANES)
      for shift in (8, 4, 2, 1):
        perm = (lane + shift) % _LANES
        acc = acc | jnp.take_along_axis(
            acc, perm, axis=0, mode="promise_in_bounds")

      @pl.when(acc[0] > 0)
      def _zero_rows():
        def zero_group(g, c2):
          idx16 = idx_v[pl.ds(g * _LANES, _LANES)]
          mf = jnp.where(idx16 == _PAD_ID, 0.0, 1.0)
          for r in range(_LANES):
            row = g * _LANES + r
            s = mf[r]
            for c in range(d // _LANES):
              vec = rows_v[row, pl.ds(c * _LANES, _LANES)]
              rows_v[row, pl.ds(c * _LANES, _LANES)] = vec * s
          return c2
        lax.fori_loop(0, _CHUNK // _LANES, zero_group, 0)

      pltpu.sync_copy(rows_v, out_hbm.at[pl.ds(off, _CHUNK)])
      return carry

    lax.fori_loop(0, steps, step, 0)

  return gather_kernel


def kernel(indices, table):
  b, l = indices.shape
  v, d = table.shape
  idx = indices.reshape(-1).astype(jnp.int32)
  out = _make_gather(b * l, d)(table, idx)
  return out.reshape(b, l, d)


# speedup vs baseline: 1.8795x; 1.1945x over previous
import functools

import jax
import jax.numpy as jnp
from jax import lax
from jax.experimental import pallas as pl
from jax.experimental.pallas import tpu as pltpu
from jax.experimental.pallas import tpu_sc as plsc

_LANES = 16
_NUM_CORES = 2
_NUM_SUBCORES = 16
_NW = _NUM_CORES * _NUM_SUBCORES
_IDXB = 128
_KSUB = 2
_CHUNK = _IDXB * _KSUB
_NBUF = 4
_LOOKAHEAD = 2
_PAD_ID = 0


def _make_gather(total_rows: int, d: int):
  assert total_rows % (_NW * _CHUNK * _NBUF) == 0
  rows_per_w = total_rows // _NW
  steps = rows_per_w // _CHUNK
  nsub = rows_per_w // _IDXB
  mesh = plsc.VectorSubcoreMesh(core_axis_name="c", subcore_axis_name="s")

  @functools.partial(
      pl.kernel,
      mesh=mesh,
      out_type=jax.ShapeDtypeStruct((total_rows, d), jnp.float32),
      scratch_types=(
          [pltpu.VMEM((nsub, _IDXB), jnp.int32)]
          + [pltpu.VMEM((_CHUNK, d), jnp.float32)] * _NBUF
          + [pltpu.SemaphoreType.DMA] * (2 * _NBUF)
      ),
      compiler_params=pltpu.CompilerParams(use_tc_tiling_on_sc=False),
  )
  def gather_kernel(table_hbm, idx_hbm, out_hbm, idx_all, *bufs_and_sems):
    rows = bufs_and_sems[:_NBUF]
    gsem = bufs_and_sems[_NBUF:2 * _NBUF]
    osem = bufs_and_sems[2 * _NBUF:]
    wid = lax.axis_index("s") * _NUM_CORES + lax.axis_index("c")
    base = wid * rows_per_w

    pltpu.sync_copy(idx_hbm.at[wid], idx_all)

    def fire_gathers(s, slot):
      for jj in range(_KSUB):
        pltpu.async_copy(
            table_hbm.at[idx_all.at[s * _KSUB + jj]],
            rows[slot].at[pl.ds(jj * _IDXB, _IDXB)],
            gsem[slot],
        )

    def drain_gathers(slot):
      for jj in range(_KSUB):
        pltpu.make_async_copy(
            table_hbm.at[idx_all.at[jj]],
            rows[slot].at[pl.ds(jj * _IDXB, _IDXB)],
            gsem[slot],
        ).wait()

    for b in range(_LOOKAHEAD):
      fire_gathers(b, b)

    izeros = jnp.zeros((_LANES,), jnp.int32)
    ones = jnp.ones((_LANES,), jnp.int32)
    lane = lax.iota(jnp.int32, _LANES)
    groups_per_sub = _IDXB // _LANES

    def outer(t0, carry):
      for b in range(_NBUF):
        s = t0 * _NBUF + b
        drain_gathers(b)

        acc = izeros
        for jj in range(_KSUB):
          for g in range(groups_per_sub):
            idx16 = idx_all[s * _KSUB + jj, pl.ds(g * _LANES, _LANES)]
            acc = acc | jnp.where(idx16 == _PAD_ID, ones, izeros)
        red = acc
        for shift in (8, 4, 2, 1):
          perm = (lane + shift) % _LANES
          red = red | jnp.take_along_axis(
              red, perm, axis=0, mode="promise_in_bounds")

        @pl.when(red[0] > 0)
        def _zero_rows():
          def zero_group(g, c2):
            jrow = s * _KSUB + g // groups_per_sub
            goff = (g % groups_per_sub) * _LANES
            idx16 = idx_all[jrow, pl.ds(goff, _LANES)]
            mf = jnp.where(idx16 == _PAD_ID, 0.0, 1.0)
            for r in range(_LANES):
              row = g * _LANES + r
              sc = mf[r]
              for c in range(d // _LANES):
                vec = rows[b][row, pl.ds(c * _LANES, _LANES)]
                rows[b][row, pl.ds(c * _LANES, _LANES)] = vec * sc
            return c2
          lax.fori_loop(0, _CHUNK // _LANES, zero_group, 0)

        pltpu.async_copy(
            rows[b], out_hbm.at[pl.ds(base + s * _CHUNK, _CHUNK)], osem[b])

        nslot = (b + _LOOKAHEAD) % _NBUF
        sn = s + _LOOKAHEAD

        @pl.when(sn < steps)
        def _fire_ahead():
          @pl.when(s >= _LOOKAHEAD)
          def _drain_store():
            pltpu.make_async_copy(
                rows[nslot],
                out_hbm.at[pl.ds(base, _CHUNK)],
                osem[nslot],
            ).wait()
          fire_gathers(sn, nslot)

      return carry

    lax.fori_loop(0, steps // _NBUF, outer, 0)

    for b in range(_NBUF):
      pltpu.make_async_copy(
          rows[b], out_hbm.at[pl.ds(base, _CHUNK)], osem[b]).wait()

  return gather_kernel


def kernel(indices, table):
  b, l = indices.shape
  v, d = table.shape
  total = b * l
  rows_per_w = total // _NW
  idx = indices.reshape(-1).astype(jnp.int32)
  idx3 = idx.reshape(_NW, rows_per_w // _IDXB, _IDXB)
  out = _make_gather(total, d)(table, idx3)
  return out.reshape(b, l, d)
